# baseline (speedup 1.0000x reference)
import functools

import jax
import jax.numpy as jnp
from jax import lax
from jax.experimental import pallas as pl
from jax.experimental.pallas import tpu as pltpu
from jax.experimental.pallas import tpu_sc as plsc

HID = 64
BATCH = 16384
HIST = 50

NUM_CORES = 2
NUM_SUBCORES = 16
NUM_WORKERS = NUM_CORES * NUM_SUBCORES
BLK = BATCH // NUM_WORKERS
BT_PER_W = BLK // 128
NK = HID // 16


_mesh = plsc.VectorSubcoreMesh(core_axis_name="c", subcore_axis_name="s")


@functools.partial(
    pl.kernel,
    out_type=jax.ShapeDtypeStruct((HIST, 8, BATCH // 128, 8, 128),
                                  jnp.float32),
    mesh=_mesh,
    scratch_types=[
        [pltpu.VMEM((1, BLK), jnp.int32) for _ in range(2)],
        [pltpu.VMEM((BLK, HID), jnp.float32) for _ in range(2)],
        pltpu.VMEM((8, BT_PER_W, 8, 128), jnp.float32),
        [pltpu.SemaphoreType.DMA for _ in range(2)],
        [pltpu.SemaphoreType.DMA for _ in range(2)],
        pltpu.SemaphoreType.DMA,
    ],
    compiler_params=pltpu.CompilerParams(use_tc_tiling_on_sc=False,
                                         needs_layout_passes=False),
)
def _gather_kernel(idxT_hbm, table_hbm, out2, iv, rows, tbuf, isem, gsem,
                   wsem):
    wid = lax.axis_index("s") * NUM_CORES + lax.axis_index("c")
    b0 = wid * BLK

    def idx_copy(h, buf):
        return pltpu.make_async_copy(
            idxT_hbm.at[pl.ds(h, 1), pl.ds(b0, BLK)], iv[buf], isem[buf])

    def gather(h_iv_buf, rows_buf):
        return pltpu.make_async_copy(
            table_hbm.at[iv[h_iv_buf].at[0]], rows[rows_buf],
            gsem[rows_buf])

    def out_copy(h):
        return pltpu.make_async_copy(
            tbuf, out2.at[h, :, pl.ds(wid * BT_PER_W, BT_PER_W)], wsem)

    idx_copy(0, 0).start()
    idx_copy(0, 0).wait()
    gather(0, 0).start()
    idx_copy(1, 1).start()

    lane = lax.iota(jnp.int32, 16)
    pat_r = [(k * 16 + lane) // 8 for k in range(NK)]
    pat_c = [lax.rem(k * 16 + lane, 8) for k in range(NK)]

    def transpose_block(rows_ref):
        for btl in range(BT_PER_W):

            btl_v = jnp.full((16,), btl, jnp.int32)

            @plsc.parallel_loop(0, 128, unroll=8)
            def _(j2):
                bsv = jnp.full((16,), j2, jnp.int32)
                for k in range(NK):
                    val = rows_ref[btl * 128 + j2, pl.ds(k * 16, 16)]
                    plsc.store_scatter(tbuf,
                                       [pat_r[k], btl_v, pat_c[k], bsv], val)

    def outer(hh, c):
        for b in range(2):
            h = hh * 2 + b
            gather(b, b).wait()

            @pl.when(h + 1 < HIST)
            def _():
                idx_copy(h + 1, 1 - b).wait()
                gather(1 - b, 1 - b).start()

            @pl.when(h + 2 < HIST)
            def _():
                idx_copy(h + 2, b).start()

            if b == 0:
                @pl.when(h > 0)
                def _():
                    out_copy(h).wait()
            else:
                out_copy(h).wait()

            transpose_block(rows[b])
            out_copy(h).start()
        return c

    lax.fori_loop(0, HIST // 2, outer, 0)
    out_copy(HIST - 1).wait()


def kernel(inputs, weight):
    out2 = _gather_kernel(inputs.T, weight)
    out5 = out2.reshape(HIST, 8, BATCH // 128, 8, 128)
    return out5.transpose(2, 4, 0, 1, 3).reshape(BATCH, HIST, HID)


# speedup vs baseline: 1.0956x; 1.0956x over previous
import functools

import jax
import jax.numpy as jnp
from jax import lax
from jax.experimental import pallas as pl
from jax.experimental.pallas import tpu as pltpu
from jax.experimental.pallas import tpu_sc as plsc

HIDDEN = 64
BATCH = 16384
HIST = 50
TOTAL = BATCH * HIST

NUM_CORES = 2
NUM_SUBCORES = 16
NUM_WORKERS = NUM_CORES * NUM_SUBCORES
PER_WORKER = TOTAL // NUM_WORKERS
CHUNK = 512
NCHUNK = PER_WORKER // CHUNK
NBUF = 2
NROUND = NCHUNK // NBUF

_mesh = plsc.VectorSubcoreMesh(core_axis_name="c", subcore_axis_name="s")


@functools.partial(
    pl.kernel,
    out_type=jax.ShapeDtypeStruct((TOTAL, HIDDEN), jnp.float32),
    mesh=_mesh,
    scratch_types=[
        pltpu.VMEM((NCHUNK, CHUNK), jnp.int32),
        [pltpu.VMEM((CHUNK, HIDDEN), jnp.float32) for _ in range(NBUF)],
        [pltpu.SemaphoreType.DMA for _ in range(NBUF)],
        [pltpu.SemaphoreType.DMA for _ in range(NBUF)],
    ],
    compiler_params=pltpu.CompilerParams(use_tc_tiling_on_sc=False),
)
def _gather_kernel(idx_hbm, table_hbm, out_hbm, idx_v, rows, g_sem, w_sem):
    wid = lax.axis_index("s") * NUM_CORES + lax.axis_index("c")
    base = wid * NCHUNK

    def out_slice(i):
        return out_hbm.at[pl.ds((base + i) * CHUNK, CHUNK)]

    pltpu.sync_copy(idx_hbm.at[pl.ds(base, NCHUNK)], idx_v)

    for b in range(NBUF):
        pltpu.async_copy(table_hbm.at[idx_v.at[b]], rows[b], g_sem[b])

    def round_body(r, carry):
        g = r * NBUF
        for b in range(NBUF):
            pltpu.make_async_copy(table_hbm.at[idx_v.at[g + b]], rows[b],
                                  g_sem[b]).wait()
            pltpu.async_copy(rows[b], out_slice(g + b), w_sem[b])
        for b in range(NBUF):
            pltpu.make_async_copy(rows[b], out_slice(g + b), w_sem[b]).wait()
            pltpu.async_copy(table_hbm.at[idx_v.at[g + NBUF + b]], rows[b],
                             g_sem[b])
        return carry

    lax.fori_loop(0, NROUND - 1, round_body, 0)

    g = (NROUND - 1) * NBUF
    for b in range(NBUF):
        pltpu.make_async_copy(table_hbm.at[idx_v.at[g + b]], rows[b],
                              g_sem[b]).wait()
        pltpu.async_copy(rows[b], out_slice(g + b), w_sem[b])
    for b in range(NBUF):
        pltpu.make_async_copy(rows[b], out_slice(g + b), w_sem[b]).wait()


def kernel(inputs, weight):
    idx = inputs.reshape(TOTAL // CHUNK, CHUNK).astype(jnp.int32)
    out = _gather_kernel(idx, weight)
    return out.reshape(inputs.shape + (HIDDEN,))
